# baseline (speedup 1.0000x reference)
import jax
import jax.numpy as jnp
from jax.experimental import pallas as pl
from jax.experimental.pallas import tpu as pltpu

_NDMA = 32


def _copy_kernel(probs_hbm, out_hbm, sems):
    rows = probs_hbm.shape[0]
    slab = rows // _NDMA
    cps = []
    for c in range(_NDMA):
        cp = pltpu.make_async_copy(
            probs_hbm.at[pl.ds(c * slab, slab)],
            out_hbm.at[pl.ds(c * slab, slab)],
            sems.at[c],
        )
        cp.start()
        cps.append(cp)
    for cp in cps:
        cp.wait()


def kernel(x, top_k_probs, top_k_indices, router_logits, w_gate, w_noise):
    t, k = top_k_probs.shape
    return pl.pallas_call(
        _copy_kernel,
        in_specs=[pl.BlockSpec(memory_space=pltpu.MemorySpace.HBM)],
        out_specs=pl.BlockSpec(memory_space=pltpu.MemorySpace.HBM),
        scratch_shapes=[pltpu.SemaphoreType.DMA((_NDMA,))],
        out_shape=jax.ShapeDtypeStruct((t, k), top_k_probs.dtype),
    )(top_k_probs)


# speedup vs baseline: 17.3903x; 17.3903x over previous
import jax
import jax.numpy as jnp
from jax.experimental import pallas as pl
from jax.experimental.pallas import tpu as pltpu

_CHUNKS = 8


def _copy_kernel(probs_hbm, out_hbm, buf_v, in_sems, out_sems):
    rows = probs_hbm.shape[0]
    chunk = rows // _CHUNKS
    ins = []
    for c in range(_CHUNKS):
        cp = pltpu.make_async_copy(
            probs_hbm.at[pl.ds(c * chunk, chunk)],
            buf_v.at[pl.ds(c * chunk, chunk)],
            in_sems.at[c],
        )
        cp.start()
        ins.append(cp)
    outs = []
    for c in range(_CHUNKS):
        ins[c].wait()
        cp = pltpu.make_async_copy(
            buf_v.at[pl.ds(c * chunk, chunk)],
            out_hbm.at[pl.ds(c * chunk, chunk)],
            out_sems.at[c],
        )
        cp.start()
        outs.append(cp)
    for c in range(_CHUNKS):
        outs[c].wait()


def kernel(x, top_k_probs, top_k_indices, router_logits, w_gate, w_noise):
    t, k = top_k_probs.shape
    return pl.pallas_call(
        _copy_kernel,
        in_specs=[pl.BlockSpec(memory_space=pltpu.MemorySpace.HBM)],
        out_specs=pl.BlockSpec(memory_space=pltpu.MemorySpace.HBM),
        scratch_shapes=[
            pltpu.VMEM((t, k), top_k_probs.dtype),
            pltpu.SemaphoreType.DMA((_CHUNKS,)),
            pltpu.SemaphoreType.DMA((_CHUNKS,)),
        ],
        out_shape=jax.ShapeDtypeStruct((t, k), top_k_probs.dtype),
    )(top_k_probs)
